# baseline (speedup 1.0000x reference)
import functools

import jax
import jax.numpy as jnp
from jax import lax
from jax.experimental import pallas as pl
from jax.experimental.pallas import tpu as pltpu
from jax.experimental.pallas import tpu_sc as plsc

SCALE = 8.0

Q = 4


@functools.lru_cache(maxsize=None)
def _build(S, P, V, D):
    info = plsc.get_sparse_core_info()
    nw = info.num_cores * info.num_subcores
    n_chunks = (S // 128) * P
    cw = n_chunks // nw
    sh_n = S // 128
    n_iters = cw // Q

    @functools.partial(
        pl.kernel,
        out_type=jax.ShapeDtypeStruct((n_chunks * (D // 8), 8, 128), jnp.float32),
        mesh=plsc.VectorSubcoreMesh(core_axis_name="c", subcore_axis_name="s"),
        compiler_params=pltpu.CompilerParams(
            use_tc_tiling_on_sc=False, needs_layout_passes=False
        ),
        scratch_types=[
            pltpu.VMEM((cw, 128), jnp.int32),
            pltpu.VMEM((Q * 128, D), jnp.float32),
            pltpu.VMEM((Q * D, 128), jnp.float32),
            pltpu.SemaphoreType.DMA,
            pltpu.SemaphoreType.DMA,
        ],
    )
    def emb(tok_hbm, table_hbm, out_hbm, idx_v, buf, sbuf, gsem, ssem):
        wid = lax.axis_index("s") * info.num_cores + lax.axis_index("c")
        gid0 = wid * cw
        pltpu.sync_copy(tok_hbm.at[pl.ds(gid0, cw)], idx_v)
        iota = lax.iota(jnp.int32, 16)
        perm = [(iota + k) & 15 for k in range(16)]

        def it_body(t, carry):
            q0 = t * Q
            gathers = [
                pltpu.async_copy(
                    table_hbm.at[idx_v.at[q0 + q]],
                    buf.at[pl.ds(q * 128, 128)],
                    gsem,
                )
                for q in range(Q)
            ]
            for g in gathers:
                g.wait()

            def sl_body(s8, c2):
                sl0 = s8 * 16
                rowq = [
                    [perm[k] + (sl0 + q * 128) for k in range(16)]
                    for q in range(Q)
                ]
                scol = [perm[k] + sl0 for k in range(16)]

                def e_body(e8, c3):
                    e0 = e8 * 16
                    lcol = iota + e0
                    for q in range(Q):
                        srow = iota + (q * D + e0)
                        for k in range(16):
                            vec = plsc.load_gather(buf, [rowq[q][k], lcol])
                            plsc.store_scatter(
                                sbuf, [srow, scol[k]], vec * SCALE
                            )
                    return c3

                lax.fori_loop(0, D // 16, e_body, 0)
                return c2

            lax.fori_loop(0, 8, sl_body, 0)

            stores = []
            for q in range(Q):
                gid = gid0 + q0 + q
                p_lo = gid % 8
                ph_sh = gid // 8
                sh = ph_sh % sh_n
                p = 8 * (ph_sh // sh_n) + p_lo
                cb = p * (D // 8) * sh_n + sh
                for g in range(D // 8):
                    stores.append(
                        pltpu.async_copy(
                            sbuf.at[pl.ds(q * D + 8 * g, 8)],
                            out_hbm.at[cb + g * sh_n],
                            ssem,
                        )
                    )
            for st in stores:
                st.wait()
            return carry

        lax.fori_loop(0, n_iters, it_body, 0)

    return emb


def kernel(tokens, table):
    S, P = tokens.shape
    V, D = table.shape
    tok2 = (
        tokens.T.reshape(P // 8, 8, S // 128, 128)
        .transpose(0, 2, 1, 3)
        .reshape((S // 128) * P, 128)
        .astype(jnp.int32)
    )
    a2 = _build(S, P, V, D)(tok2, table)
    out = (
        a2.reshape(P, D // 8, S // 128, 8, 128)
        .transpose(2, 4, 0, 1, 3)
        .reshape(S, P, D)
    )
    return out


# speedup vs baseline: 1.4054x; 1.4054x over previous
import functools

import jax
import jax.numpy as jnp
from jax import lax
from jax.experimental import pallas as pl
from jax.experimental.pallas import tpu as pltpu
from jax.experimental.pallas import tpu_sc as plsc

SCALE = 8.0

Q = 4


@functools.lru_cache(maxsize=None)
def _build(S, P, V, D):
    info = plsc.get_sparse_core_info()
    nw = info.num_cores * info.num_subcores
    n_chunks = (S // 128) * P
    cw = n_chunks // nw
    sh_n = S // 128
    n_iters = cw // Q

    @functools.partial(
        pl.kernel,
        out_type=jax.ShapeDtypeStruct((n_chunks * (D // 8), 8, 128), jnp.float32),
        mesh=plsc.VectorSubcoreMesh(core_axis_name="c", subcore_axis_name="s"),
        compiler_params=pltpu.CompilerParams(
            use_tc_tiling_on_sc=False, needs_layout_passes=False
        ),
        scratch_types=[
            pltpu.VMEM((cw, 128), jnp.int32),
            pltpu.VMEM((Q * 128, D), jnp.float32),
            pltpu.VMEM((Q * D, 128), jnp.float32),
            pltpu.SemaphoreType.DMA,
            pltpu.SemaphoreType.DMA,
        ],
    )
    def emb(tok_hbm, table_hbm, out_hbm, idx_v, buf, sbuf, gsem, ssem):
        wid = lax.axis_index("s") * info.num_cores + lax.axis_index("c")
        gid0 = wid * cw
        pltpu.sync_copy(tok_hbm.at[pl.ds(gid0, cw)], idx_v)
        iota = lax.iota(jnp.int32, 16)
        perm = [(iota + k) & 15 for k in range(16)]

        def it_body(t, carry):
            q0 = t * Q
            gathers = [
                pltpu.async_copy(
                    table_hbm.at[idx_v.at[q0 + q]],
                    buf.at[pl.ds(q * 128, 128)],
                    gsem,
                )
                for q in range(Q)
            ]
            for g in gathers:
                g.wait()

            def sl_body(s8, c2):
                sl0 = s8 * 16

                def e_body(e8, c3):
                    e0 = e8 * 16
                    lcol = iota + e0
                    for q in range(Q):
                        srow = iota + (q * D + e0)
                        for kb in range(0, 16, 8):
                            vecs = []
                            for k in range(kb, kb + 8):
                                rowv = perm[k] + (sl0 + q * 128)
                                vecs.append(
                                    plsc.load_gather(buf, [rowv, lcol])
                                )
                            for j, k in enumerate(range(kb, kb + 8)):
                                scol = perm[k] + sl0
                                plsc.store_scatter(
                                    sbuf, [srow, scol], vecs[j] * SCALE
                                )
                    return c3

                lax.fori_loop(0, D // 16, e_body, 0)
                return c2

            lax.fori_loop(0, 8, sl_body, 0)

            stores = []
            for q in range(Q):
                gid = gid0 + q0 + q
                p_lo = gid % 8
                ph_sh = gid // 8
                sh = ph_sh % sh_n
                p = 8 * (ph_sh // sh_n) + p_lo
                cb = p * (D // 8) * sh_n + sh
                for g in range(D // 8):
                    stores.append(
                        pltpu.async_copy(
                            sbuf.at[pl.ds(q * D + 8 * g, 8)],
                            out_hbm.at[cb + g * sh_n],
                            ssem,
                        )
                    )
            for st in stores:
                st.wait()
            return carry

        lax.fori_loop(0, n_iters, it_body, 0)

    return emb


def kernel(tokens, table):
    S, P = tokens.shape
    V, D = table.shape
    tok2 = (
        tokens.T.reshape(P // 8, 8, S // 128, 128)
        .transpose(0, 2, 1, 3)
        .reshape((S // 128) * P, 128)
        .astype(jnp.int32)
    )
    a2 = _build(S, P, V, D)(tok2, table)
    out = (
        a2.reshape(P, D // 8, S // 128, 8, 128)
        .transpose(2, 4, 0, 1, 3)
        .reshape(S, P, D)
    )
    return out


# speedup vs baseline: 1.4088x; 1.0024x over previous
import functools

import jax
import jax.numpy as jnp
from jax import lax
from jax.experimental import pallas as pl
from jax.experimental.pallas import tpu as pltpu
from jax.experimental.pallas import tpu_sc as plsc

SCALE = 8.0

Q = 4


@functools.lru_cache(maxsize=None)
def _build(S, P, V, D):
    info = plsc.get_sparse_core_info()
    nw = info.num_cores * info.num_subcores
    n_chunks = (S // 128) * P
    cw = n_chunks // nw
    sh_n = S // 128
    n_iters = cw // Q

    @functools.partial(
        pl.kernel,
        out_type=jax.ShapeDtypeStruct((n_chunks * (D // 8), 8, 128), jnp.float32),
        mesh=plsc.VectorSubcoreMesh(core_axis_name="c", subcore_axis_name="s"),
        compiler_params=pltpu.CompilerParams(
            use_tc_tiling_on_sc=False, needs_layout_passes=False
        ),
        scratch_types=[
            pltpu.VMEM((cw, 128), jnp.int32),
            pltpu.VMEM((Q * 128, D), jnp.float32),
            pltpu.VMEM((Q * D, 128), jnp.float32),
            pltpu.SemaphoreType.DMA,
            pltpu.SemaphoreType.DMA,
        ],
    )
    def emb(tok_hbm, table_hbm, out_hbm, idx_v, buf, sbuf, gsem, ssem):
        wid = lax.axis_index("s") * info.num_cores + lax.axis_index("c")
        gid0 = wid * cw
        pltpu.sync_copy(tok_hbm.at[pl.ds(gid0, cw)], idx_v)
        iota = lax.iota(jnp.int32, 16)
        perm = [(iota + k) & 15 for k in range(16)]

        def it_body(t, carry):
            q0 = t * Q
            gathers = [
                pltpu.async_copy(
                    table_hbm.at[idx_v.at[q0 + q]],
                    buf.at[pl.ds(q * 128, 128)],
                    gsem,
                )
                for q in range(Q)
            ]
            for g in gathers:
                g.wait()

            def sl_body(s8, c2):
                sl0 = s8 * 16

                def e_body(e8, c3):
                    e0 = e8 * 16
                    lcol = iota + e0
                    for q in range(Q):
                        srow = iota + (q * D + e0)
                        for kb in range(0, 16, 8):
                            vecs = []
                            for k in range(kb, kb + 8):
                                rowv = perm[k] + (sl0 + q * 128)
                                vecs.append(
                                    plsc.load_gather(buf, [rowv, lcol])
                                )
                            for j, k in enumerate(range(kb, kb + 8)):
                                scol = perm[k] + sl0
                                plsc.store_scatter(
                                    sbuf, [srow, scol], vecs[j] * SCALE
                                )
                    return c3

                lax.fori_loop(0, D // 16, e_body, 0)
                return c2

            lax.fori_loop(0, 8, sl_body, 0)

            stores = []
            for q in range(Q):
                gid = gid0 + q0 + q
                p_lo = gid % 8
                ph_sh = gid // 8
                sh = ph_sh % sh_n
                p = 8 * (ph_sh // sh_n) + p_lo
                cb = p * (D // 8) * sh_n + sh
                for g in range(D // 8):
                    stores.append(
                        pltpu.async_copy(
                            sbuf.at[pl.ds(q * D + 8 * g, 8)],
                            out_hbm.at[cb + g * sh_n],
                            ssem,
                        )
                    )
            for st in stores:
                st.wait()
            return carry

        lax.fori_loop(0, n_iters, it_body, 0)

    return emb


def _detile_body(i_ref, o_ref):
    x = i_ref[...]
    s_n = x.shape[1] // 128
    o_ref[...] = x.reshape(8, s_n, 128).transpose(1, 0, 2).reshape(8 * s_n, 128)


@functools.lru_cache(maxsize=None)
def _detile_build(S, P):
    return pl.pallas_call(
        _detile_body,
        grid=(P // 8,),
        in_specs=[pl.BlockSpec((8, S), lambda i: (i, 0))],
        out_specs=pl.BlockSpec((8 * (S // 128), 128), lambda i: (i, 0)),
        out_shape=jax.ShapeDtypeStruct(((S // 128) * P, 128), jnp.int32),
    )


def kernel(tokens, table):
    S, P = tokens.shape
    V, D = table.shape
    tok2 = _detile_build(S, P)(tokens.T.astype(jnp.int32))
    a2 = _build(S, P, V, D)(tok2, table)
    out = (
        a2.reshape(P, D // 8, S // 128, 8, 128)
        .transpose(2, 4, 0, 1, 3)
        .reshape(S, P, D)
    )
    return out


# speedup vs baseline: 1.5119x; 1.0732x over previous
import functools

import jax
import jax.numpy as jnp
from jax import lax
from jax.experimental import pallas as pl
from jax.experimental.pallas import tpu as pltpu
from jax.experimental.pallas import tpu_sc as plsc

SCALE = 8.0

Q = 4


@functools.lru_cache(maxsize=None)
def _build(S, P, V, D):
    info = plsc.get_sparse_core_info()
    nw = info.num_cores * info.num_subcores
    n_chunks = (S // 128) * P
    cw = n_chunks // nw
    sh_n = S // 128
    n_iters = cw // Q

    @functools.partial(
        pl.kernel,
        out_type=jax.ShapeDtypeStruct((n_chunks * (D // 8), 8, 128), jnp.float32),
        mesh=plsc.VectorSubcoreMesh(core_axis_name="c", subcore_axis_name="s"),
        compiler_params=pltpu.CompilerParams(
            use_tc_tiling_on_sc=False, needs_layout_passes=False
        ),
        scratch_types=[
            pltpu.VMEM((cw, 128), jnp.int32),
            pltpu.VMEM((Q * 128, D), jnp.float32),
            pltpu.VMEM((Q * D, 128), jnp.float32),
            pltpu.SemaphoreType.DMA,
            pltpu.SemaphoreType.DMA,
        ],
    )
    def emb(tok_hbm, table_hbm, out_hbm, idx_v, buf, sbuf, gsem, ssem):
        wid = lax.axis_index("s") * info.num_cores + lax.axis_index("c")
        gid0 = wid * cw
        pltpu.sync_copy(tok_hbm.at[pl.ds(gid0, cw)], idx_v)
        iota = lax.iota(jnp.int32, 16)
        perm = [(iota + k) & 15 for k in range(16)]

        def it_body(t, carry):
            q0 = t * Q
            gathers = [
                pltpu.async_copy(
                    table_hbm.at[idx_v.at[q0 + q]],
                    buf.at[pl.ds(q * 128, 128)],
                    gsem,
                )
                for q in range(Q)
            ]
            for g in gathers:
                g.wait()

            def sl_body(s8, c2):
                sl0 = s8 * 16

                def e_body(e8, c3):
                    e0 = e8 * 16
                    lcol = iota + e0
                    for q in range(Q):
                        srow = iota + (q * D + e0)
                        for kb in range(0, 16, 8):
                            vecs = []
                            for k in range(kb, kb + 8):
                                rowv = perm[k] + (sl0 + q * 128)
                                vecs.append(
                                    plsc.load_gather(buf, [rowv, lcol])
                                )
                            for j, k in enumerate(range(kb, kb + 8)):
                                scol = perm[k] + sl0
                                plsc.store_scatter(
                                    sbuf, [srow, scol], vecs[j] * SCALE
                                )
                    return c3

                lax.fori_loop(0, D // 16, e_body, 0)
                return c2

            lax.fori_loop(0, 8, sl_body, 0)

            stores = []
            for q in range(Q):
                gid = gid0 + q0 + q
                p_lo = gid % 8
                ph_sh = gid // 8
                sh = ph_sh % sh_n
                p = 8 * (ph_sh // sh_n) + p_lo
                cb = p * (D // 8) * sh_n + sh
                for g in range(D // 8):
                    stores.append(
                        pltpu.async_copy(
                            sbuf.at[pl.ds(q * D + 8 * g, 8)],
                            out_hbm.at[cb + g * sh_n],
                            ssem,
                        )
                    )
            for st in stores:
                st.wait()
            return carry

        lax.fori_loop(0, n_iters, it_body, 0)

    return emb


def _detile_body(i_ref, o_ref):
    x = i_ref[...]
    s_n = x.shape[1] // 128
    o_ref[...] = (
        x.reshape(8, s_n, 128).transpose(1, 0, 2).reshape(8 * s_n, 128) * 2
    )


@functools.lru_cache(maxsize=None)
def _detile_build(S, P):
    return pl.pallas_call(
        _detile_body,
        grid=(P // 8,),
        in_specs=[pl.BlockSpec((8, S), lambda i: (i, 0))],
        out_specs=pl.BlockSpec((8 * (S // 128), 128), lambda i: (i, 0)),
        out_shape=jax.ShapeDtypeStruct(((S // 128) * P, 128), jnp.int32),
    )


def kernel(tokens, table):
    S, P = tokens.shape
    V, D = table.shape
    tok2 = _detile_build(S, P)(tokens.T.astype(jnp.int32))
    tpad = jnp.pad(table, ((0, 0), (0, 128 - D))).reshape(2 * V, D)
    a2 = _build(S, P, 2 * V, D)(tok2, tpad)
    out = (
        a2.reshape(P, D // 8, S // 128, 8, 128)
        .transpose(2, 4, 0, 1, 3)
        .reshape(S, P, D)
    )
    return out


# speedup vs baseline: 1.5477x; 1.0237x over previous
import functools

import jax
import jax.numpy as jnp
from jax import lax
from jax.experimental import pallas as pl
from jax.experimental.pallas import tpu as pltpu
from jax.experimental.pallas import tpu_sc as plsc

SCALE = 8.0

Q = 4


@functools.lru_cache(maxsize=None)
def _build(S, P, V, D):
    info = plsc.get_sparse_core_info()
    nw = info.num_cores * info.num_subcores
    n_chunks = (S // 128) * P
    cw = n_chunks // nw
    sh_n = S // 128
    n_pairs = cw // (2 * Q)

    @functools.partial(
        pl.kernel,
        out_type=jax.ShapeDtypeStruct((n_chunks * (D // 8), 8, 128), jnp.float32),
        mesh=plsc.VectorSubcoreMesh(core_axis_name="c", subcore_axis_name="s"),
        compiler_params=pltpu.CompilerParams(
            use_tc_tiling_on_sc=False, needs_layout_passes=False
        ),
        scratch_types=[
            pltpu.VMEM((cw, 128), jnp.int32),
            pltpu.VMEM((Q * 128, D), jnp.float32),
            pltpu.VMEM((Q * 128, D), jnp.float32),
            pltpu.VMEM((Q * D, 128), jnp.float32),
            pltpu.SemaphoreType.DMA,
            pltpu.SemaphoreType.DMA,
            pltpu.SemaphoreType.DMA,
        ],
    )
    def emb(
        tok_hbm, table_hbm, out_hbm, idx_v, buf_a, buf_b, sbuf,
        gsem_a, gsem_b, ssem,
    ):
        wid = lax.axis_index("s") * info.num_cores + lax.axis_index("c")
        gid0 = wid * cw
        pltpu.sync_copy(tok_hbm.at[pl.ds(gid0, cw)], idx_v)
        iota = lax.iota(jnp.int32, 16)
        perm = [(iota + k) & 15 for k in range(16)]

        def gfire(buf, gsem, t):
            return [
                pltpu.async_copy(
                    table_hbm.at[idx_v.at[t * Q + q]],
                    buf.at[pl.ds(q * 128, 128)],
                    gsem,
                )
                for q in range(Q)
            ]

        def transpose(buf):
            def sl_body(s8, c2):
                sl0 = s8 * 16

                def e_body(e8, c3):
                    e0 = e8 * 16
                    lcol = iota + e0
                    for q in range(Q):
                        srow = iota + (q * D + e0)
                        for kb in range(0, 16, 8):
                            vecs = []
                            for k in range(kb, kb + 8):
                                rowv = perm[k] + (sl0 + q * 128)
                                vecs.append(
                                    plsc.load_gather(buf, [rowv, lcol])
                                )
                            for j, k in enumerate(range(kb, kb + 8)):
                                scol = perm[k] + sl0
                                plsc.store_scatter(
                                    sbuf, [srow, scol], vecs[j] * SCALE
                                )
                    return c3

                lax.fori_loop(0, D // 16, e_body, 0)
                return c2

            lax.fori_loop(0, 8, sl_body, 0)

        def flush(t):
            stores = []
            for q in range(Q):
                gid = gid0 + t * Q + q
                p_lo = gid % 8
                ph_sh = gid // 8
                sh = ph_sh % sh_n
                p = 8 * (ph_sh // sh_n) + p_lo
                cb = p * (D // 8) * sh_n + sh
                for g in range(D // 8):
                    stores.append(
                        pltpu.async_copy(
                            sbuf.at[pl.ds(q * D + 8 * g, 8)],
                            out_hbm.at[cb + g * sh_n],
                            ssem,
                        )
                    )
            for st in stores:
                st.wait()

        def it_body(tt, carry):
            t0 = 2 * tt
            t1 = t0 + 1
            ga = gfire(buf_a, gsem_a, t0)
            gb = gfire(buf_b, gsem_b, t1)
            for g in ga:
                g.wait()
            transpose(buf_a)
            flush(t0)
            for g in gb:
                g.wait()
            transpose(buf_b)
            flush(t1)
            return carry

        lax.fori_loop(0, n_pairs, it_body, 0)

    return emb


def _detile_body(i_ref, o_ref):
    x = i_ref[...]
    s_n = x.shape[1] // 128
    o_ref[...] = (
        x.reshape(8, s_n, 128).transpose(1, 0, 2).reshape(8 * s_n, 128) * 2
    )


@functools.lru_cache(maxsize=None)
def _detile_build(S, P):
    return pl.pallas_call(
        _detile_body,
        grid=(P // 8,),
        in_specs=[pl.BlockSpec((8, S), lambda i: (i, 0))],
        out_specs=pl.BlockSpec((8 * (S // 128), 128), lambda i: (i, 0)),
        out_shape=jax.ShapeDtypeStruct(((S // 128) * P, 128), jnp.int32),
    )


def kernel(tokens, table):
    S, P = tokens.shape
    V, D = table.shape
    tok2 = _detile_build(S, P)(tokens.T.astype(jnp.int32))
    tpad = jnp.pad(table, ((0, 0), (0, 128 - D))).reshape(2 * V, D)
    a2 = _build(S, P, 2 * V, D)(tok2, tpad)
    out = (
        a2.reshape(P, D // 8, S // 128, 8, 128)
        .transpose(2, 4, 0, 1, 3)
        .reshape(S, P, D)
    )
    return out
